# baseline (speedup 1.0000x reference)
import functools

import jax
import jax.numpy as jnp
from jax import lax
from jax.experimental import pallas as pl
from jax.experimental.pallas import tpu as pltpu
from jax.experimental.pallas import tpu_sc as plsc

NC = 2
NS = 16


def _sc_aggregate(x, src3, dst3, zeros_blk, chunk):
    n, d = x.shape
    _, n_chunks, _, batch = dst3.shape
    n_iter = n_chunks * chunk
    rows_per_tile = zeros_blk.shape[0]
    n_pad = rows_per_tile * NS
    K = chunk

    mesh = plsc.VectorSubcoreMesh(
        core_axis_name="c", subcore_axis_name="s", num_cores=NC, num_subcores=NS
    )

    @functools.partial(
        pl.kernel,
        out_type=jax.ShapeDtypeStruct((NC, n_pad, d), jnp.float32),
        mesh=mesh,
        scratch_types=[
            pltpu.VMEM((2, K, batch), jnp.int32),
            pltpu.VMEM((2, K, batch), jnp.int32),
            pltpu.VMEM((batch, d), jnp.float32),
            pltpu.VMEM((batch, d), jnp.float32),
            pltpu.VMEM((batch, d), jnp.float32),
            pltpu.VMEM_SHARED((n_pad, d), jnp.float32),
            pltpu.SemaphoreType.DMA,
            pltpu.SemaphoreType.DMA,
            pltpu.SemaphoreType.DMA,
            pltpu.SemaphoreType.DMA,
            pltpu.SemaphoreType.DMA,
            pltpu.SemaphoreType.DMA,
            pltpu.SemaphoreType.DMA,
        ],
    )
    def agg_kernel(x_hbm, src_hbm, dst_hbm, zeros_hbm, out_hbm,
                   src_v, dst_v, b0, b1, b2, acc_sh,
                   g0, g1, g2, s0, s1, s2, isem):
        c = lax.axis_index("c")
        s = lax.axis_index("s")
        wid = c * NS + s
        row0 = s * rows_per_tile

        pltpu.sync_copy(zeros_hbm, acc_sh.at[pl.ds(row0, rows_per_tile)])
        pltpu.sync_copy(src_hbm.at[wid, 0], src_v.at[0])
        pltpu.sync_copy(dst_hbm.at[wid, 0], dst_v.at[0])

        plsc.subcore_barrier()

        def refill_wait(slot):
            pltpu.make_async_copy(src_hbm.at[wid, 0],
                                  src_v.at[slot], isem).wait()
            pltpu.make_async_copy(dst_hbm.at[wid, 0],
                                  dst_v.at[slot], isem).wait()

        def gather(i, buf, sem):
            slot = (i // K) % 2
            k = i % K
            pltpu.async_copy(x_hbm.at[src_v.at[slot, k]], buf, sem)

        def gather_managed(i, buf, sem):
            @pl.when(i % K == 0)
            def _():
                refill_wait((i // K) % 2)

            @pl.when(jnp.logical_and(i % K == 3, i + K - 3 < n_iter))
            def _():
                nxt = i // K + 1
                pltpu.async_copy(src_hbm.at[wid, nxt], src_v.at[nxt % 2],
                                 isem)
                pltpu.async_copy(dst_hbm.at[wid, nxt], dst_v.at[nxt % 2],
                                 isem)

            gather(i, buf, sem)

        def gather_wait(i, buf, sem):
            slot = (i // K) % 2
            k = i % K
            pltpu.make_async_copy(x_hbm.at[src_v.at[slot, k]], buf, sem).wait()

        def scat(i, buf, sem):
            slot = (i // K) % 2
            k = i % K
            pltpu.async_copy(buf, acc_sh.at[dst_v.at[slot, k]], sem, add=True)

        def scat_wait(buf, sem):
            pltpu.make_async_copy(buf, acc_sh.at[dst_v.at[0, 0]], sem).wait()

        gather(0, b0, g0)
        gather(1, b1, g1)
        gather(2, b2, g2)

        @pl.loop(0, n_iter - 2, step=3)
        def _edges(j):
            gather_wait(j, b0, g0)
            scat(j, b0, s0)
            gather_wait(j + 1, b1, g1)
            scat(j + 1, b1, s1)
            scat_wait(b0, s0)

            @pl.when(j + 3 < n_iter)
            def _():
                gather_managed(j + 3, b0, g0)

            gather_wait(j + 2, b2, g2)
            scat(j + 2, b2, s2)
            scat_wait(b1, s1)

            @pl.when(j + 4 < n_iter)
            def _():
                gather_managed(j + 4, b1, g1)

            scat_wait(b2, s2)

            @pl.when(j + 5 < n_iter)
            def _():
                gather_managed(j + 5, b2, g2)

        if n_iter % 3 == 2:
            gather_wait(n_iter - 2, b0, g0)
            pltpu.sync_copy(b0, acc_sh.at[dst_v.at[((n_iter - 2) // K) % 2,
                                                   (n_iter - 2) % K]],
                            add=True)
            gather_wait(n_iter - 1, b1, g1)
            pltpu.sync_copy(b1, acc_sh.at[dst_v.at[((n_iter - 1) // K) % 2,
                                                   (n_iter - 1) % K]],
                            add=True)
        elif n_iter % 3 == 1:
            gather_wait(n_iter - 1, b0, g0)
            pltpu.sync_copy(b0, acc_sh.at[dst_v.at[((n_iter - 1) // K) % 2,
                                                   (n_iter - 1) % K]],
                            add=True)

        plsc.subcore_barrier()
        pltpu.sync_copy(acc_sh.at[pl.ds(row0, rows_per_tile)],
                        out_hbm.at[c, pl.ds(row0, rows_per_tile)])

    return agg_kernel(x, src3, dst3, zeros_blk)


def _mlp_body(x_ref, a0_ref, a1_ref, wa_ref, ba_ref, wb_ref, bb_ref, o_ref):
    h = x_ref[...] + a0_ref[...] + a1_ref[...]
    h = jnp.maximum(
        jnp.dot(h, wa_ref[...], preferred_element_type=jnp.float32)
        + ba_ref[...], 0.0)
    h = jnp.maximum(
        jnp.dot(h, wb_ref[...], preferred_element_type=jnp.float32)
        + bb_ref[...], 0.0)
    o_ref[...] = h


def _mlp2_body(x_ref, a0_ref, a1_ref, wa_ref, ba_ref, wb_ref, bb_ref,
               wfc_ref, bfc_ref, o_ref):
    h = x_ref[...] + a0_ref[...] + a1_ref[...]
    h = jnp.maximum(
        jnp.dot(h, wa_ref[...], preferred_element_type=jnp.float32)
        + ba_ref[...], 0.0)
    h = jnp.maximum(
        jnp.dot(h, wb_ref[...], preferred_element_type=jnp.float32)
        + bb_ref[...], 0.0)
    logits = (jnp.dot(h, wfc_ref[...], preferred_element_type=jnp.float32)
              + bfc_ref[...])
    m = jnp.max(logits, axis=1, keepdims=True)
    lse = jnp.log(jnp.sum(jnp.exp(logits - m), axis=1, keepdims=True)) + m
    o_ref[...] = logits - lse


def _specs(bn, d, n_mats):
    row = pl.BlockSpec((bn, d), lambda i: (i, 0))
    mat = pl.BlockSpec((d, d), lambda i: (0, 0))
    vec = pl.BlockSpec((1, d), lambda i: (0, 0))
    return [row, row, row] + [mat, vec] * n_mats


def _mlp(x, a0, a1, wa, ba, wb, bb, bn):
    n, d = x.shape
    return pl.pallas_call(
        _mlp_body,
        grid=(n // bn,),
        in_specs=_specs(bn, d, 2),
        out_specs=pl.BlockSpec((bn, d), lambda i: (i, 0)),
        out_shape=jax.ShapeDtypeStruct((n, d), jnp.float32),
    )(x, a0, a1, wa, ba.reshape(1, d), wb, bb.reshape(1, d))


def _mlp2_fc_logsoftmax(x, a0, a1, wa, ba, wb, bb, wfc_p, bfc_p, bn):
    n, d = x.shape
    return pl.pallas_call(
        _mlp2_body,
        grid=(n // bn,),
        in_specs=_specs(bn, d, 3),
        out_specs=pl.BlockSpec((bn, d), lambda i: (i, 0)),
        out_shape=jax.ShapeDtypeStruct((n, d), jnp.float32),
    )(x, a0, a1, wa, ba.reshape(1, d), wb, bb.reshape(1, d),
      wfc_p, bfc_p.reshape(1, d))


def kernel(x, edge_index, W1a, b1a, W1b, b1b, W2a, b2a, W2b, b2b, Wfc, bfc):
    n, d = x.shape
    e = edge_index.shape[1]
    c = Wfc.shape[1]

    nt = NC * NS
    e_per_w = e // nt
    batch = 80
    n_iter = e_per_w // batch
    chunk = 25
    src3 = edge_index[0].reshape(nt, n_iter // chunk, chunk, batch)
    dst3 = edge_index[1].reshape(nt, n_iter // chunk, chunk, batch)
    n_pad = -(-n // (NS * 8)) * (NS * 8)
    zeros_blk = jnp.zeros((n_pad // NS, d), jnp.float32)

    wfc_p = jnp.zeros((d, d), jnp.float32).at[:, :c].set(Wfc)
    bfc_p = jnp.full((d,), -1e30, jnp.float32).at[:c].set(bfc)

    bn = 1000
    agg1 = _sc_aggregate(x, src3, dst3, zeros_blk, chunk)
    h1 = _mlp(x, agg1[0], agg1[1], W1a, b1a, W1b, b1b, bn)
    agg2 = _sc_aggregate(h1, src3, dst3, zeros_blk, chunk)
    out_p = _mlp2_fc_logsoftmax(h1, agg2[0], agg2[1], W2a, b2a, W2b, b2b,
                                wfc_p, bfc_p, bn)
    return out_p[:, :c]
